# baseline (speedup 1.0000x reference)
import functools

import jax
import jax.numpy as jnp
from jax import lax
from jax.experimental import pallas as pl
from jax.experimental.pallas import tpu as pltpu
from jax.experimental.pallas import tpu_sc as plsc


def _build_sc_call(S, B, F, L):
    info = plsc.get_sparse_core_info()
    NC, NS = info.num_cores, info.num_subcores
    NW = NC * NS
    rows_w = B // NW
    R = min(rows_w, 128)
    CH = rows_w // R
    SP = 32
    D = 6

    mesh = plsc.VectorSubcoreMesh(core_axis_name="c", subcore_axis_name="s")

    @functools.partial(
        pl.kernel,
        mesh=mesh,
        out_type=jax.ShapeDtypeStruct((S * B, L), jnp.float32),
        scratch_types=[
            pltpu.VMEM((SP,), jnp.int32),
            *([pltpu.VMEM((R, L), jnp.float32)] * D),
            *([pltpu.SemaphoreType.DMA] * (2 * D)),
        ],
    )
    def fused_slice(inp, slices, out, slv, *bufs_sems):
        bufs = bufs_sems[:D]
        isems = bufs_sems[D:2 * D]
        osems = bufs_sems[2 * D:]
        wid = lax.axis_index("s") * NC + lax.axis_index("c")
        base = wid * rows_w

        pltpu.sync_copy(slices, slv)
        parts = [slv[pl.ds(p * 16, 16)] for p in range(SP // 16)]
        offs = [pl.multiple_of(parts[s // 16][s % 16], L) for s in range(S)]

        tasks = [(s, c) for c in range(CH) for s in range(S)]
        T = len(tasks)

        def start_in(t):
            s, c = tasks[t]
            return pltpu.async_copy(
                inp.at[pl.ds(base + c * R, R), pl.ds(offs[s], L)],
                bufs[t % D],
                isems[t % D],
            )

        def start_out(t):
            s, c = tasks[t]
            return pltpu.async_copy(
                bufs[t % D],
                out.at[pl.ds(s * B + base + c * R, R)],
                osems[t % D],
            )

        in_h = [None] * T
        out_h = [None] * T
        for t in range(min(D - 1, T)):
            in_h[t] = start_in(t)
        for t in range(T):
            in_h[t].wait()
            out_h[t] = start_out(t)
            if t + D - 1 < T:
                if t - 1 >= 0:
                    out_h[t - 1].wait()
                in_h[t + D - 1] = start_in(t + D - 1)
        for t in range(max(T - D, 0), T):
            out_h[t].wait()

    return fused_slice


def kernel(input_tensor, slices_index, slice_len):
    B, F = input_tensor.shape
    S = slices_index.shape[0]
    L = F // S
    sl_pad = jnp.zeros((32,), jnp.int32).at[:S].set(slices_index.astype(jnp.int32))
    out2d = _build_sc_call(S, B, F, L)(input_tensor, sl_pad)
    return out2d.reshape(S, B, L)


# speedup vs baseline: 1.1089x; 1.1089x over previous
import functools

import jax
import jax.numpy as jnp
from jax import lax
from jax.experimental import pallas as pl
from jax.experimental.pallas import tpu as pltpu
from jax.experimental.pallas import tpu_sc as plsc


def _build_sc_call(S, B, F, L):
    info = plsc.get_sparse_core_info()
    NC, NS = info.num_cores, info.num_subcores
    NW = NC * NS
    rows_w = B // NW
    R = min(rows_w, 256)
    CH = rows_w // R
    SP = 32
    D = 3

    mesh = plsc.VectorSubcoreMesh(core_axis_name="c", subcore_axis_name="s")

    @functools.partial(
        pl.kernel,
        mesh=mesh,
        out_type=jax.ShapeDtypeStruct((S * B, L), jnp.float32),
        scratch_types=[
            pltpu.VMEM((SP,), jnp.int32),
            *([pltpu.VMEM((R, L), jnp.float32)] * D),
            *([pltpu.SemaphoreType.DMA] * (2 * D)),
        ],
    )
    def fused_slice(inp, slices, out, slv, *bufs_sems):
        bufs = bufs_sems[:D]
        isems = bufs_sems[D:2 * D]
        osems = bufs_sems[2 * D:]
        wid = lax.axis_index("s") * NC + lax.axis_index("c")
        base = wid * rows_w

        pltpu.sync_copy(slices, slv)
        parts = [slv[pl.ds(p * 16, 16)] for p in range(SP // 16)]
        offs = [pl.multiple_of(parts[s // 16][s % 16], L) for s in range(S)]

        tasks = [(s, c) for s in range(S) for c in range(CH)]
        T = len(tasks)

        def start_in(t):
            s, c = tasks[t]
            return pltpu.async_copy(
                inp.at[pl.ds(base + c * R, R), pl.ds(offs[s], L)],
                bufs[t % D],
                isems[t % D],
            )

        def start_out(t):
            s, c = tasks[t]
            return pltpu.async_copy(
                bufs[t % D],
                out.at[pl.ds(s * B + base + c * R, R)],
                osems[t % D],
            )

        in_h = [None] * T
        out_h = [None] * T
        for t in range(min(D - 1, T)):
            in_h[t] = start_in(t)
        for t in range(T):
            in_h[t].wait()
            out_h[t] = start_out(t)
            if t + D - 1 < T:
                if t - 1 >= 0:
                    out_h[t - 1].wait()
                in_h[t + D - 1] = start_in(t + D - 1)
        for t in range(max(T - D, 0), T):
            out_h[t].wait()

    return fused_slice


def kernel(input_tensor, slices_index, slice_len):
    B, F = input_tensor.shape
    S = slices_index.shape[0]
    L = F // S
    sl_pad = jnp.zeros((32,), jnp.int32).at[:S].set(slices_index.astype(jnp.int32))
    out2d = _build_sc_call(S, B, F, L)(input_tensor, sl_pad)
    return out2d.reshape(S, B, L)
